# baseline (speedup 1.0000x reference)
import functools
import math

import jax
import jax.numpy as jnp
from jax import lax
from jax.experimental import pallas as pl
from jax.experimental.pallas import tpu as pltpu
from jax.experimental.pallas import tpu_sc as plsc

NC = 2
NS = 16
NW = NC * NS
CHUNK = 64
NB = 5
CNT_W = 128
ZROWS = 8
HALVES = 8
CNT_Q = 5


def _zero_shared(z_v, sh, row0, rows_per_sub, sem):
  nz = rows_per_sub // ZROWS
  rem = rows_per_sub - nz * ZROWS
  for t in range(nz):
    pltpu.async_copy(z_v, sh.at[pl.ds(row0 + ZROWS * t, ZROWS)], sem)
  if rem:
    pltpu.async_copy(z_v.at[pl.ds(0, rem)],
                     sh.at[pl.ds(row0 + ZROWS * nz, rem)], sem)
  for t in range(nz):
    pltpu.make_async_copy(z_v, sh.at[pl.ds(row0, ZROWS)], sem).wait()
  if rem:
    pltpu.make_async_copy(z_v.at[pl.ds(0, rem)],
                          sh.at[pl.ds(row0, rem)], sem).wait()


def _sc_aggregate(src_chunks, dst_chunks, x_src, acc_rows, cpw, n_real):
  n_nodes, d_feat = x_src.shape
  rows_per_sub = acc_rows // NS
  d_lanes = d_feat // 16
  grp = cpw // HALVES

  mesh = plsc.VectorSubcoreMesh(core_axis_name="c", subcore_axis_name="s")

  @functools.partial(
      pl.kernel,
      mesh=mesh,
      out_type=jax.ShapeDtypeStruct((NC, acc_rows, d_feat), jnp.float32),
      scratch_types=[
          pltpu.VMEM((grp, CHUNK), jnp.int32),
          pltpu.VMEM((grp, CHUNK), jnp.int32),
          pltpu.VMEM((NB, CHUNK, d_feat), jnp.float32),
          pltpu.VMEM((ZROWS, d_feat), jnp.float32),
          pltpu.VMEM_SHARED((acc_rows, d_feat), jnp.float32),
          pltpu.SemaphoreType.DMA,
          pltpu.SemaphoreType.DMA,
          pltpu.SemaphoreType.DMA,
          pltpu.SemaphoreType.DMA,
          pltpu.SemaphoreType.DMA,
      ],
  )
  def sc_kernel(srcc_hbm, dstc_hbm, xsrc_hbm, sum_out,
                srcs_v, dsts_v, rows_v, zrow_v, acc_sh,
                sem_g0, sem_g1, sem_g2, sem_g3, sem_g4):
    c = lax.axis_index("c")
    s = lax.axis_index("s")
    wid = c * NS + s

    zeros16 = jnp.zeros((16,), jnp.float32)
    for r in range(ZROWS):
      for l in range(d_lanes):
        zrow_v[r, pl.ds(16 * l, 16)] = zeros16

    row0 = s * rows_per_sub
    _zero_shared(zrow_v, acc_sh, row0, rows_per_sub, sem_g0)
    plsc.subcore_barrier()

    bufs = [rows_v.at[k] for k in range(NB)]
    sems = [sem_g0, sem_g1, sem_g2, sem_g3, sem_g4]

    def real(j_global):
      return j_global * NW + wid < n_real

    def gather(j, buf, sem):
      pltpu.async_copy(xsrc_hbm.at[srcs_v.at[j]], buf, sem)

    def gather_wait(buf, sem):
      pltpu.make_async_copy(xsrc_hbm.at[srcs_v.at[0]], buf, sem).wait()

    nt = grp // NB

    for h in range(HALVES):
      base = h * grp

      def body(t, carry, base=base):
        for k in range(NB):
          j = NB * t + k

          @pl.when(real(base + j))
          def _():
            gather_wait(bufs[k], sems[k])
            pltpu.sync_copy(bufs[k], acc_sh.at[dsts_v.at[j]], add=True)

          @pl.when(jnp.logical_and(j + NB < grp, real(base + j + NB)))
          def _():
            gather(j + NB, bufs[k], sems[k])

        return carry

      pltpu.sync_copy(srcc_hbm.at[pl.ds(base, grp), wid], srcs_v)
      pltpu.sync_copy(dstc_hbm.at[pl.ds(base, grp), wid], dsts_v)
      for k in range(NB):
        @pl.when(real(base + k))
        def _(k=k):
          gather(k, bufs[k], sems[k])
      lax.fori_loop(0, nt, body, 0)
    plsc.subcore_barrier()

    pltpu.sync_copy(acc_sh.at[pl.ds(row0, rows_per_sub)],
                    sum_out.at[c, pl.ds(row0, rows_per_sub)])

  return sc_kernel(src_chunks, dst_chunks, x_src)


def _sc_counts(dst_chunks, acc_rows, cpw, n_real):
  rows_per_sub = acc_rows // NS

  mesh = plsc.VectorSubcoreMesh(core_axis_name="c", subcore_axis_name="s")

  @functools.partial(
      pl.kernel,
      mesh=mesh,
      out_type=jax.ShapeDtypeStruct((NC, acc_rows, CNT_W), jnp.float32),
      scratch_types=[
          pltpu.VMEM((cpw, CHUNK), jnp.int32),
          pltpu.VMEM((CHUNK, CNT_W), jnp.float32),
          pltpu.VMEM((ZROWS, CNT_W), jnp.float32),
          pltpu.VMEM_SHARED((acc_rows, CNT_W), jnp.float32),
          pltpu.SemaphoreType.DMA,
      ],
  )
  def cnt_kernel(dstc_hbm, cnt_out, dsts_v, ones_v, zcnt_v, cnt_sh, sem):
    c = lax.axis_index("c")
    s = lax.axis_index("s")
    wid = c * NS + s

    zeros16 = jnp.zeros((16,), jnp.float32)
    ones16 = jnp.ones((16,), jnp.float32)
    cnt_lanes = CNT_W // 16
    for r in range(ZROWS):
      for l in range(cnt_lanes):
        zcnt_v[r, pl.ds(16 * l, 16)] = zeros16
    for r in range(CHUNK):
      for l in range(cnt_lanes):
        ones_v[r, pl.ds(16 * l, 16)] = ones16

    pltpu.sync_copy(dstc_hbm.at[:, wid], dsts_v)

    row0 = s * rows_per_sub
    _zero_shared(zcnt_v, cnt_sh, row0, rows_per_sub, sem)
    plsc.subcore_barrier()

    def real(j):
      return j * NW + wid < n_real

    def group_body(g, carry):
      for q in range(CNT_Q):
        j = g * CNT_Q + q

        @pl.when(real(j))
        def _():
          pltpu.async_copy(ones_v, cnt_sh.at[dsts_v.at[j]], sem, add=True)

      for q in range(CNT_Q):
        j = g * CNT_Q + q

        @pl.when(real(j))
        def _():
          pltpu.make_async_copy(ones_v, cnt_sh.at[dsts_v.at[0]],
                                sem).wait()

      return carry

    lax.fori_loop(0, cpw // CNT_Q, group_body, 0)
    plsc.subcore_barrier()

    pltpu.sync_copy(cnt_sh.at[pl.ds(row0, rows_per_sub)],
                    cnt_out.at[c, pl.ds(row0, rows_per_sub)])

  return cnt_kernel(dst_chunks)


def _tc_finalize(x_dst_pad, sums, cnts, w_t, b2, block_rows):
  acc_rows, d_feat = x_dst_pad.shape
  out_ch = w_t.shape[1]

  def tc_kernel(xd_ref, p_ref, c_ref, wt_ref, b_ref, o_ref):
    p = p_ref[0] + p_ref[1]
    cnt = c_ref[0, :, 0:1] + c_ref[1, :, 0:1]
    agg = p / jnp.maximum(cnt, 1.0)
    wt = wt_ref[...]
    h = jnp.dot(xd_ref[...], wt[:d_feat], precision=lax.Precision.HIGHEST)
    h = h + jnp.dot(agg, wt[d_feat:], precision=lax.Precision.HIGHEST)
    o_ref[...] = jnp.maximum(h + b_ref[...], 0.0)

  return pl.pallas_call(
      tc_kernel,
      grid=(acc_rows // block_rows,),
      in_specs=[
          pl.BlockSpec((block_rows, d_feat), lambda i: (i, 0)),
          pl.BlockSpec((NC, block_rows, d_feat), lambda i: (0, i, 0)),
          pl.BlockSpec((NC, block_rows, CNT_W), lambda i: (0, i, 0)),
          pl.BlockSpec((2 * d_feat, out_ch), lambda i: (0, 0)),
          pl.BlockSpec((1, out_ch), lambda i: (0, 0)),
      ],
      out_specs=pl.BlockSpec((block_rows, out_ch), lambda i: (i, 0)),
      out_shape=jax.ShapeDtypeStruct((acc_rows, out_ch), jnp.float32),
  )(x_dst_pad, sums, cnts, w_t, b2)


def kernel(x_src, x_dst, edge_index, W, b):
  n_nodes, d_feat = x_src.shape
  n_edges = edge_index.shape[1]

  src = edge_index[0].astype(jnp.int32)
  dst = edge_index[1].astype(jnp.int32)

  n_real = -(-n_edges // CHUNK)
  cpw = -(-n_real // NW)
  q = math.lcm(NB * HALVES, CNT_Q)
  cpw = -(-cpw // q) * q
  tail = cpw * NW * CHUNK - n_edges
  if tail:
    src = jnp.concatenate([src, jnp.zeros((tail,), jnp.int32)])
    dst = jnp.concatenate([dst, jnp.full((tail,), n_nodes, jnp.int32)])
  src_chunks = src.reshape(cpw, NW, CHUNK)
  dst_chunks = dst.reshape(cpw, NW, CHUNK)

  acc_rows = -(-(n_nodes + 1) // (8 * NS)) * (8 * NS)

  sums = _sc_aggregate(src_chunks, dst_chunks, x_src, acc_rows, cpw, n_real)
  cnts = _sc_counts(dst_chunks, acc_rows, cpw, n_real)

  block_rows = 8
  for d in range(2048, 7, -8):
    if n_nodes % d == 0:
      block_rows = d
      break
  return _tc_finalize(x_dst, sums, cnts, W.T, b.reshape(1, -1), block_rows)


# speedup vs baseline: 1.0866x; 1.0866x over previous
import functools
import math

import jax
import jax.numpy as jnp
from jax import lax
from jax.experimental import pallas as pl
from jax.experimental.pallas import tpu as pltpu
from jax.experimental.pallas import tpu_sc as plsc

NC = 2
NS = 16
NW = NC * NS
CHUNK = 64
NB = 4
CNT_W = 128
ZROWS = 8
HALVES = 3
CNT_Q = 6


def _zero_shared(z_v, sh, row0, rows_per_sub, sem):
  nz = rows_per_sub // ZROWS
  rem = rows_per_sub - nz * ZROWS
  for t in range(nz):
    pltpu.async_copy(z_v, sh.at[pl.ds(row0 + ZROWS * t, ZROWS)], sem)
  if rem:
    pltpu.async_copy(z_v.at[pl.ds(0, rem)],
                     sh.at[pl.ds(row0 + ZROWS * nz, rem)], sem)
  for t in range(nz):
    pltpu.make_async_copy(z_v, sh.at[pl.ds(row0, ZROWS)], sem).wait()
  if rem:
    pltpu.make_async_copy(z_v.at[pl.ds(0, rem)],
                          sh.at[pl.ds(row0, rem)], sem).wait()


def _sc_aggregate(src_chunks, dst_chunks, x_src, acc_rows, cpw, n_real):
  n_nodes, d_feat = x_src.shape
  rows_per_sub = acc_rows // NS
  d_lanes = d_feat // 16
  grp = cpw // HALVES

  mesh = plsc.VectorSubcoreMesh(core_axis_name="c", subcore_axis_name="s")

  @functools.partial(
      pl.kernel,
      mesh=mesh,
      out_type=jax.ShapeDtypeStruct((NC, acc_rows, d_feat), jnp.float32),
      scratch_types=[
          pltpu.VMEM((grp, CHUNK), jnp.int32),
          pltpu.VMEM((grp, CHUNK), jnp.int32),
          pltpu.VMEM((NB, CHUNK, d_feat), jnp.float32),
          pltpu.VMEM((ZROWS, d_feat), jnp.float32),
          pltpu.VMEM_SHARED((acc_rows, d_feat), jnp.float32),
          pltpu.SemaphoreType.DMA,
          pltpu.SemaphoreType.DMA,
          pltpu.SemaphoreType.DMA,
          pltpu.SemaphoreType.DMA,
      ],
  )
  def sc_kernel(srcc_hbm, dstc_hbm, xsrc_hbm, sum_out,
                srcs_v, dsts_v, rows_v, zrow_v, acc_sh,
                sem_g0, sem_g1, sem_g2, sem_g3):
    c = lax.axis_index("c")
    s = lax.axis_index("s")
    wid = c * NS + s

    zeros16 = jnp.zeros((16,), jnp.float32)
    for r in range(ZROWS):
      for l in range(d_lanes):
        zrow_v[r, pl.ds(16 * l, 16)] = zeros16

    row0 = s * rows_per_sub
    _zero_shared(zrow_v, acc_sh, row0, rows_per_sub, sem_g0)
    plsc.subcore_barrier()

    bufs = [rows_v.at[k] for k in range(NB)]
    sems = [sem_g0, sem_g1, sem_g2, sem_g3]

    def real(j_global):
      return j_global * NW + wid < n_real

    def gather(j, buf, sem):
      pltpu.async_copy(xsrc_hbm.at[srcs_v.at[j]], buf, sem)

    def gather_wait(buf, sem):
      pltpu.make_async_copy(xsrc_hbm.at[srcs_v.at[0]], buf, sem).wait()

    nt = grp // NB

    for h in range(HALVES):
      base = h * grp

      def body(t, carry, base=base):
        for k in range(NB):
          j = NB * t + k

          @pl.when(real(base + j))
          def _():
            gather_wait(bufs[k], sems[k])
            pltpu.sync_copy(bufs[k], acc_sh.at[dsts_v.at[j]], add=True)

          @pl.when(jnp.logical_and(j + NB < grp, real(base + j + NB)))
          def _():
            gather(j + NB, bufs[k], sems[k])

        return carry

      pltpu.sync_copy(srcc_hbm.at[pl.ds(base, grp), wid], srcs_v)
      pltpu.sync_copy(dstc_hbm.at[pl.ds(base, grp), wid], dsts_v)
      for k in range(NB):
        @pl.when(real(base + k))
        def _(k=k):
          gather(k, bufs[k], sems[k])
      lax.fori_loop(0, nt, body, 0)
    plsc.subcore_barrier()

    pltpu.sync_copy(acc_sh.at[pl.ds(row0, rows_per_sub)],
                    sum_out.at[c, pl.ds(row0, rows_per_sub)])

  return sc_kernel(src_chunks, dst_chunks, x_src)


def _sc_counts(dst_chunks, acc_rows, cpw, n_real):
  rows_per_sub = acc_rows // NS

  mesh = plsc.VectorSubcoreMesh(core_axis_name="c", subcore_axis_name="s")

  @functools.partial(
      pl.kernel,
      mesh=mesh,
      out_type=jax.ShapeDtypeStruct((NC, acc_rows, CNT_W), jnp.float32),
      scratch_types=[
          pltpu.VMEM((cpw, CHUNK), jnp.int32),
          pltpu.VMEM((CHUNK, CNT_W), jnp.float32),
          pltpu.VMEM((ZROWS, CNT_W), jnp.float32),
          pltpu.VMEM_SHARED((acc_rows, CNT_W), jnp.float32),
          pltpu.SemaphoreType.DMA,
      ],
  )
  def cnt_kernel(dstc_hbm, cnt_out, dsts_v, ones_v, zcnt_v, cnt_sh, sem):
    c = lax.axis_index("c")
    s = lax.axis_index("s")
    wid = c * NS + s

    zeros16 = jnp.zeros((16,), jnp.float32)
    ones16 = jnp.ones((16,), jnp.float32)
    cnt_lanes = CNT_W // 16
    for r in range(ZROWS):
      for l in range(cnt_lanes):
        zcnt_v[r, pl.ds(16 * l, 16)] = zeros16
    for r in range(CHUNK):
      for l in range(cnt_lanes):
        ones_v[r, pl.ds(16 * l, 16)] = ones16

    pltpu.sync_copy(dstc_hbm.at[:, wid], dsts_v)

    row0 = s * rows_per_sub
    _zero_shared(zcnt_v, cnt_sh, row0, rows_per_sub, sem)
    plsc.subcore_barrier()

    def real(j):
      return j * NW + wid < n_real

    def group_body(g, carry):
      for q in range(CNT_Q):
        j = g * CNT_Q + q

        @pl.when(real(j))
        def _():
          pltpu.async_copy(ones_v, cnt_sh.at[dsts_v.at[j]], sem, add=True)

      for q in range(CNT_Q):
        j = g * CNT_Q + q

        @pl.when(real(j))
        def _():
          pltpu.make_async_copy(ones_v, cnt_sh.at[dsts_v.at[0]],
                                sem).wait()

      return carry

    lax.fori_loop(0, cpw // CNT_Q, group_body, 0)
    plsc.subcore_barrier()

    pltpu.sync_copy(cnt_sh.at[pl.ds(row0, rows_per_sub)],
                    cnt_out.at[c, pl.ds(row0, rows_per_sub)])

  return cnt_kernel(dst_chunks)


def _tc_pre(x_dst, w1_t, b2, block_rows):
  n_rows, d_feat = x_dst.shape
  out_ch = w1_t.shape[1]

  def tc_kernel(xd_ref, wt_ref, b_ref, o_ref):
    o_ref[...] = jnp.dot(xd_ref[...], wt_ref[...],
                         precision=lax.Precision.HIGHEST) + b_ref[...]

  return pl.pallas_call(
      tc_kernel,
      grid=(n_rows // block_rows,),
      in_specs=[
          pl.BlockSpec((block_rows, d_feat), lambda i: (i, 0)),
          pl.BlockSpec((d_feat, out_ch), lambda i: (0, 0)),
          pl.BlockSpec((1, out_ch), lambda i: (0, 0)),
      ],
      out_specs=pl.BlockSpec((block_rows, out_ch), lambda i: (i, 0)),
      out_shape=jax.ShapeDtypeStruct((n_rows, out_ch), jnp.float32),
  )(x_dst, w1_t, b2)


def _tc_finalize(h1, sums, cnts, w2_t, block_rows):
  n_rows, out_ch = h1.shape
  d_feat = w2_t.shape[0]

  def tc_kernel(h1_ref, p_ref, c_ref, wt_ref, o_ref):
    p = p_ref[0] + p_ref[1]
    cnt = c_ref[0, :, 0:1] + c_ref[1, :, 0:1]
    agg = p / jnp.maximum(cnt, 1.0)
    h = h1_ref[...] + jnp.dot(agg, wt_ref[...],
                              precision=lax.Precision.HIGHEST)
    o_ref[...] = jnp.maximum(h, 0.0)

  return pl.pallas_call(
      tc_kernel,
      grid=(n_rows // block_rows,),
      in_specs=[
          pl.BlockSpec((block_rows, out_ch), lambda i: (i, 0)),
          pl.BlockSpec((NC, block_rows, d_feat), lambda i: (0, i, 0)),
          pl.BlockSpec((NC, block_rows, CNT_W), lambda i: (0, i, 0)),
          pl.BlockSpec((d_feat, out_ch), lambda i: (0, 0)),
      ],
      out_specs=pl.BlockSpec((block_rows, out_ch), lambda i: (i, 0)),
      out_shape=jax.ShapeDtypeStruct((n_rows, out_ch), jnp.float32),
  )(h1, sums, cnts, w2_t)


def kernel(x_src, x_dst, edge_index, W, b):
  n_nodes, d_feat = x_src.shape
  n_edges = edge_index.shape[1]

  src = edge_index[0].astype(jnp.int32)
  dst = edge_index[1].astype(jnp.int32)

  n_real = -(-n_edges // CHUNK)
  cpw = -(-n_real // NW)
  q = math.lcm(NB * HALVES, CNT_Q)
  cpw = -(-cpw // q) * q
  tail = cpw * NW * CHUNK - n_edges
  if tail:
    src = jnp.concatenate([src, jnp.zeros((tail,), jnp.int32)])
    dst = jnp.concatenate([dst, jnp.full((tail,), n_nodes, jnp.int32)])
  src_chunks = src.reshape(cpw, NW, CHUNK)
  dst_chunks = dst.reshape(cpw, NW, CHUNK)

  acc_rows = -(-(n_nodes + 1) // (8 * NS)) * (8 * NS)

  sums = _sc_aggregate(src_chunks, dst_chunks, x_src, acc_rows, cpw, n_real)
  cnts = _sc_counts(dst_chunks, acc_rows, cpw, n_real)

  block_rows = 8
  for d in range(2048, 7, -8):
    if n_nodes % d == 0:
      block_rows = d
      break
  w_t = W.T
  h1 = _tc_pre(x_dst, w_t[:d_feat], b.reshape(1, -1), block_rows)
  return _tc_finalize(h1, sums, cnts, w_t[d_feat:], block_rows)
